# baseline (speedup 1.0000x reference)
import functools

import jax
import jax.numpy as jnp
from jax import lax
from jax.experimental import pallas as pl
from jax.experimental.pallas import tpu as pltpu
from jax.experimental.pallas import tpu_sc as plsc

_LANES = 16
_IDX_COLS = 128
_CHUNK = 128


def _make_kernel(B, D, NC, NS):
    NW = NC * NS
    BW = B // NW
    NIDX = BW // _IDX_COLS
    NCHUNK = BW // _CHUNK
    NGRP = _CHUNK // _LANES
    W2 = 2 * D
    mesh = plsc.VectorSubcoreMesh(core_axis_name="c", subcore_axis_name="s")

    @functools.partial(
        pl.kernel,
        mesh=mesh,
        out_type=jax.ShapeDtypeStruct((B,), jnp.float32),
        compiler_params=pltpu.CompilerParams(needs_layout_passes=False),
        scratch_types=[
            pltpu.VMEM((NIDX, _IDX_COLS), jnp.int32),
            pltpu.VMEM((NIDX, _IDX_COLS), jnp.int32),
            pltpu.VMEM((BW,), jnp.int32),
            pltpu.VMEM((BW,), jnp.int32),
            pltpu.VMEM((2, _CHUNK, 128), jnp.float32),
            pltpu.VMEM((2, _CHUNK, 128), jnp.float32),
            pltpu.VMEM((BW,), jnp.float32),
            pltpu.SemaphoreType.DMA,
            pltpu.SemaphoreType.DMA,
        ],
    )
    def k(user_hbm, item_hbm, uemb_hbm, iemb_hbm, out_hbm,
          ublk, iblk, ucb, icb, ubuf, ibuf, outv, sem0, sem1):
        wid = lax.axis_index("s") * NC + lax.axis_index("c")
        base = wid * BW
        sems = (sem0, sem1)

        for c in range(NIDX):
            pltpu.sync_copy(
                user_hbm.at[pl.ds(base + c * _IDX_COLS, _IDX_COLS)],
                ublk.at[c])
            pltpu.sync_copy(
                item_hbm.at[pl.ds(base + c * _IDX_COLS, _IDX_COLS)],
                iblk.at[c])
        for c in range(NIDX):
            for t in range(_IDX_COLS // _LANES):
                sl = pl.ds(t * _LANES, _LANES)
                fl = pl.ds(c * _IDX_COLS + t * _LANES, _LANES)
                u = ublk[c, sl]
                i = iblk[c, sl]
                ucb[fl] = (u & 1) * D
                icb[fl] = (i & 1) * D
                ublk[c, sl] = u >> 1
                iblk[c, sl] = i >> 1

        def issue(chunk, p):
            cu = pltpu.async_copy(
                uemb_hbm.at[ublk.at[chunk]], ubuf.at[p], sems[p])
            ci = pltpu.async_copy(
                iemb_hbm.at[iblk.at[chunk]], ibuf.at[p], sems[p])
            return cu, ci

        lane = lax.iota(jnp.int32, _LANES)
        pend = issue(0, 0)
        for chunk in range(NCHUNK):
            p = chunk & 1
            cu, ci = pend
            if chunk + 1 < NCHUNK:
                pend = issue(chunk + 1, 1 - p)
            cu.wait()
            ci.wait()

            def grp(g, carry, chunk=chunk, p=p):
                rows = g * _LANES + lane
                cbu = ucb[pl.ds(chunk * _CHUNK + g * _LANES, _LANES)]
                cbi = icb[pl.ds(chunk * _CHUNK + g * _LANES, _LANES)]
                acc = jnp.zeros((_LANES,), jnp.float32)
                for j in range(D):
                    uv = plsc.load_gather(ubuf.at[p], [rows, cbu + j])
                    iv = plsc.load_gather(ibuf.at[p], [rows, cbi + j])
                    acc = acc + uv * iv
                outv[pl.ds(chunk * _CHUNK + g * _LANES, _LANES)] = acc
                return carry

            lax.fori_loop(0, NGRP, grp, 0)

        pltpu.sync_copy(outv, out_hbm.at[pl.ds(base, BW)])

    return k


@jax.jit
def kernel(user, item, user_emb, item_emb):
    B = user.shape[0]
    D = user_emb.shape[1]
    info = plsc.get_sparse_core_info()
    k = _make_kernel(B, D, info.num_cores, info.num_subcores)
    uemb2 = user_emb.reshape(user_emb.shape[0] // 2, 2 * D)
    iemb2 = item_emb.reshape(item_emb.shape[0] // 2, 2 * D)
    return k(user.astype(jnp.int32), item.astype(jnp.int32), uemb2, iemb2)


# speedup vs baseline: 1.0734x; 1.0734x over previous
import functools

import jax
import jax.numpy as jnp
from jax import lax
from jax.experimental import pallas as pl
from jax.experimental.pallas import tpu as pltpu
from jax.experimental.pallas import tpu_sc as plsc

_LANES = 16
_CHUNK = 128


def _make_kernel(B, D, NC, NS):
    NW = NC * NS
    BW = B // NW
    NCHUNK = BW // _CHUNK
    NGRP = _CHUNK // _LANES
    CW = _CHUNK * D
    mesh = plsc.VectorSubcoreMesh(core_axis_name="c", subcore_axis_name="s")

    @functools.partial(
        pl.kernel,
        mesh=mesh,
        out_type=jax.ShapeDtypeStruct((B,), jnp.float32),
        compiler_params=pltpu.CompilerParams(needs_layout_passes=False),
        scratch_types=[
            pltpu.VMEM((BW,), jnp.int32),
            pltpu.VMEM((BW,), jnp.int32),
            pltpu.VMEM((2 * CW,), jnp.float32),
            pltpu.VMEM((2 * CW,), jnp.float32),
            pltpu.VMEM((BW,), jnp.float32),
            pltpu.SemaphoreType.DMA,
            pltpu.SemaphoreType.DMA,
        ],
    )
    def k(user_hbm, item_hbm, uemb_hbm, iemb_hbm, out_hbm,
          usm, ism, ubuf, ibuf, outv, sem0, sem1):
        wid = lax.axis_index("s") * NC + lax.axis_index("c")
        base = wid * BW
        sems = (sem0, sem1)

        pltpu.sync_copy(user_hbm.at[pl.ds(base, BW)], usm)
        pltpu.sync_copy(item_hbm.at[pl.ds(base, BW)], ism)

        def issue(chunk, p):
            def row16(g, carry):
                uvec = usm[pl.ds(chunk * _CHUNK + g * _LANES, _LANES)] * D
                ivec = ism[pl.ds(chunk * _CHUNK + g * _LANES, _LANES)] * D
                for r in range(_LANES):
                    off = p * CW + g * _LANES * D + r * D
                    pltpu.async_copy(
                        uemb_hbm.at[pl.ds(pl.multiple_of(uvec[r], D), D)],
                        ubuf.at[pl.ds(off, D)], sems[p])
                    pltpu.async_copy(
                        iemb_hbm.at[pl.ds(pl.multiple_of(ivec[r], D), D)],
                        ibuf.at[pl.ds(off, D)], sems[p])
                return carry
            lax.fori_loop(0, NGRP, row16, 0)

        def drain(p):
            pltpu.make_async_copy(
                out_hbm.at[pl.ds(0, CW)], ubuf.at[pl.ds(p * CW, CW)],
                sems[p]).wait()
            pltpu.make_async_copy(
                out_hbm.at[pl.ds(0, CW)], ibuf.at[pl.ds(p * CW, CW)],
                sems[p]).wait()

        lane = lax.iota(jnp.int32, _LANES)
        issue(0, 0)
        for chunk in range(NCHUNK):
            p = chunk & 1
            if chunk + 1 < NCHUNK:
                issue(chunk + 1, 1 - p)
            drain(p)

            def grp(g, carry, chunk=chunk, p=p):
                addr = (g * _LANES + lane) * D + p * CW
                acc = jnp.zeros((_LANES,), jnp.float32)
                for j in range(D):
                    uv = plsc.load_gather(ubuf, [addr + j])
                    iv = plsc.load_gather(ibuf, [addr + j])
                    acc = acc + uv * iv
                outv[pl.ds(chunk * _CHUNK + g * _LANES, _LANES)] = acc
                return carry

            lax.fori_loop(0, NGRP, grp, 0)

        pltpu.sync_copy(outv, out_hbm.at[pl.ds(base, BW)])

    return k


@jax.jit
def kernel(user, item, user_emb, item_emb):
    B = user.shape[0]
    D = user_emb.shape[1]
    info = plsc.get_sparse_core_info()
    k = _make_kernel(B, D, info.num_cores, info.num_subcores)
    return k(user.astype(jnp.int32), item.astype(jnp.int32),
             user_emb.reshape(-1), item_emb.reshape(-1))


# speedup vs baseline: 1.8194x; 1.6950x over previous
import functools

import jax
import jax.numpy as jnp
from jax import lax
from jax.experimental import pallas as pl
from jax.experimental.pallas import tpu as pltpu
from jax.experimental.pallas import tpu_sc as plsc

_LANES = 16
_CHUNK = 128


def _make_kernel(B, D, NC, NS):
    NW = NC * NS
    BW = B // NW
    NCHUNK = BW // _CHUNK
    NGRP = _CHUNK // _LANES
    mesh = plsc.VectorSubcoreMesh(core_axis_name="c", subcore_axis_name="s")

    @functools.partial(
        pl.kernel,
        mesh=mesh,
        out_type=jax.ShapeDtypeStruct((B,), jnp.float32),
        compiler_params=pltpu.CompilerParams(needs_layout_passes=False),
        scratch_types=[
            pltpu.VMEM((BW,), jnp.int32),
            pltpu.VMEM((BW,), jnp.int32),
            pltpu.VMEM((2, _CHUNK, 64), jnp.float32),
            pltpu.VMEM((2, _CHUNK, 64), jnp.float32),
            pltpu.VMEM((BW,), jnp.float32),
            pltpu.SemaphoreType.DMA,
            pltpu.SemaphoreType.DMA,
        ],
    )
    def k(user_hbm, item_hbm, uemb_hbm, iemb_hbm, out_hbm,
          usm, ism, ubuf, ibuf, outv, sem0, sem1):
        wid = lax.axis_index("s") * NC + lax.axis_index("c")
        base = wid * BW
        sems = (sem0, sem1)

        pltpu.sync_copy(user_hbm.at[pl.ds(base, BW)], usm)
        pltpu.sync_copy(item_hbm.at[pl.ds(base, BW)], ism)

        def issue(chunk, p):
            def row16(g, carry):
                uvec = usm[pl.ds(chunk * _CHUNK + g * _LANES, _LANES)]
                ivec = ism[pl.ds(chunk * _CHUNK + g * _LANES, _LANES)]
                for r in range(_LANES):
                    lr = g * _LANES + r
                    pltpu.async_copy(
                        uemb_hbm.at[pl.ds(uvec[r], 1)],
                        ubuf.at[p].at[pl.ds(lr, 1)], sems[p])
                    pltpu.async_copy(
                        iemb_hbm.at[pl.ds(ivec[r], 1)],
                        ibuf.at[p].at[pl.ds(lr, 1)], sems[p])
                return carry
            lax.fori_loop(0, NGRP, row16, 0)

        def drain(p):
            pltpu.make_async_copy(
                uemb_hbm.at[pl.ds(0, _CHUNK)], ubuf.at[p], sems[p]).wait()
            pltpu.make_async_copy(
                iemb_hbm.at[pl.ds(0, _CHUNK)], ibuf.at[p], sems[p]).wait()

        lane = lax.iota(jnp.int32, _LANES)
        issue(0, 0)
        for chunk in range(NCHUNK):
            p = chunk & 1
            if chunk + 1 < NCHUNK:
                issue(chunk + 1, 1 - p)
            drain(p)

            def grp(g, carry, chunk=chunk, p=p):
                acc = jnp.zeros((_LANES,), jnp.float32)
                for r in range(_LANES):
                    lr = g * _LANES + r
                    s = jnp.zeros((_LANES,), jnp.float32)
                    for c in range(D // _LANES):
                        sl = pl.ds(c * _LANES, _LANES)
                        s = s + ubuf[p, lr, sl] * ibuf[p, lr, sl]
                    acc = jnp.where(lane == r, jnp.sum(s), acc)
                outv[pl.ds(chunk * _CHUNK + g * _LANES, _LANES)] = acc
                return carry

            lax.fori_loop(0, NGRP, grp, 0)

        pltpu.sync_copy(outv, out_hbm.at[pl.ds(base, BW)])

    return k


@jax.jit
def kernel(user, item, user_emb, item_emb):
    B = user.shape[0]
    D = user_emb.shape[1]
    info = plsc.get_sparse_core_info()
    k = _make_kernel(B, D, info.num_cores, info.num_subcores)
    return k(user.astype(jnp.int32), item.astype(jnp.int32),
             user_emb, item_emb)
